# baseline (speedup 1.0000x reference)
import functools

import jax
import jax.numpy as jnp
from jax import lax
from jax.experimental import pallas as pl
from jax.experimental.pallas import tpu as pltpu
from jax.experimental.pallas import tpu_sc as plsc

_B, _N, _SF, _H = 2, 10000, 20, 128
_L = _N * _SF
_EPS = 1e-5

_BL = 8000
_BLN = _BL // _SF
_NBE = _L // _BL

_NC, _NS = 2, 16
_NW = _NC * _NS
_CHUNK = 128
_M = _B * _L
_GRAN = _NW * 2 * _CHUNK
_MP = ((_M + _GRAN - 1) // _GRAN) * _GRAN


def _mm(a, w):
    return lax.dot_general(a, w, (((1,), (1,)), ((), ())),
                           preferred_element_type=jnp.float32)


def _sc_gather(table, idx):
    mp, = idx.shape
    _, h = table.shape
    dt = table.dtype
    m_per_w = mp // _NW
    nsteps = m_per_w // (2 * _CHUNK)
    mesh = plsc.VectorSubcoreMesh(core_axis_name="c", subcore_axis_name="s")

    @functools.partial(
        pl.kernel,
        mesh=mesh,
        out_type=jax.ShapeDtypeStruct((mp, h), dt),
        scratch_types=[
            pltpu.VMEM((m_per_w,), jnp.int32),
            pltpu.VMEM((_CHUNK, h), dt),
            pltpu.VMEM((_CHUNK, h), dt),
            pltpu.SemaphoreType.DMA,
            pltpu.SemaphoreType.DMA,
        ],
    )
    def k(table_hbm, idx_hbm, out_hbm, idx_v, buf0, buf1, sem0, sem1):
        wid = lax.axis_index("s") * _NC + lax.axis_index("c")
        base = wid * m_per_w
        pltpu.sync_copy(idx_hbm.at[pl.ds(base, m_per_w)], idx_v)

        def body(s, carry):
            c0 = 2 * s
            c1 = c0 + 1
            g0 = pltpu.async_copy(
                table_hbm.at[idx_v.at[pl.ds(c0 * _CHUNK, _CHUNK)]], buf0, sem0)
            g1 = pltpu.async_copy(
                table_hbm.at[idx_v.at[pl.ds(c1 * _CHUNK, _CHUNK)]], buf1, sem1)
            g0.wait()
            pltpu.sync_copy(buf0, out_hbm.at[pl.ds(base + c0 * _CHUNK, _CHUNK)])
            g1.wait()
            pltpu.sync_copy(buf1, out_hbm.at[pl.ds(base + c1 * _CHUNK, _CHUNK)])
            return carry

        lax.fori_loop(0, nsteps, body, 0)

    return k(table, idx)


def _vx_body(x_ref, w_ref, b_ref, o_ref):
    o_ref[...] = _mm(x_ref[...], w_ref[...]) + b_ref[...]


def _k_vx(x_flat, w, b2):
    rows = 2000
    return pl.pallas_call(
        _vx_body,
        grid=(x_flat.shape[0] // rows,),
        in_specs=[
            pl.BlockSpec((rows, _H), lambda i: (i, 0)),
            pl.BlockSpec((_H, _H), lambda i: (0, 0)),
            pl.BlockSpec((1, _H), lambda i: (0, 0)),
        ],
        out_specs=pl.BlockSpec((rows, _H), lambda i: (i, 0)),
        out_shape=jax.ShapeDtypeStruct(x_flat.shape, jnp.float32),
    )(x_flat, w, b2)


def _stage_a_body(e_ref, vxg_ref, x_ref, we_ref, be_ref, wn_ref, bn_ref,
                  xtmp_ref, st_ref):
    b = pl.program_id(0)
    i = pl.program_id(1)
    ve = _mm(e_ref[0], we_ref[...]) + be_ref[...]
    ve3 = ve.reshape(_BLN, _SF, _H)
    m = jnp.max(ve3, axis=1, keepdims=True)
    p = jnp.exp(ve3 - m)
    s = jnp.sum(p, axis=1)
    vxg3 = vxg_ref[0].reshape(_BLN, _SF, _H)
    to = jnp.sum(p * vxg3, axis=1) / s
    xt = _mm(x_ref[0], wn_ref[...]) + bn_ref[...] + to
    xtmp_ref[0] = xt

    @pl.when(jnp.logical_and(b == 0, i == 0))
    def _():
        st_ref[...] = jnp.zeros_like(st_ref)

    st_ref[0:1, :] += jnp.sum(xt, axis=0, keepdims=True)
    st_ref[1:2, :] += jnp.sum(xt * xt, axis=0, keepdims=True)


def _stage_a(e, vxg, x, we, be2, wn, bn2):
    return pl.pallas_call(
        _stage_a_body,
        grid=(_B, _NBE),
        in_specs=[
            pl.BlockSpec((1, _BL, _H), lambda b, i: (b, i, 0)),
            pl.BlockSpec((1, _BL, _H), lambda b, i: (b, i, 0)),
            pl.BlockSpec((1, _BLN, _H), lambda b, i: (b, i, 0)),
            pl.BlockSpec((_H, _H), lambda b, i: (0, 0)),
            pl.BlockSpec((1, _H), lambda b, i: (0, 0)),
            pl.BlockSpec((_H, _H), lambda b, i: (0, 0)),
            pl.BlockSpec((1, _H), lambda b, i: (0, 0)),
        ],
        out_specs=[
            pl.BlockSpec((1, _BLN, _H), lambda b, i: (b, i, 0)),
            pl.BlockSpec((2, _H), lambda b, i: (0, 0)),
        ],
        out_shape=[
            jax.ShapeDtypeStruct((_B, _N, _H), jnp.float32),
            jax.ShapeDtypeStruct((2, _H), jnp.float32),
        ],
    )(e, vxg, x, we, be2, wn, bn2)


def _node_fin_body(x_ref, xt_ref, st_ref, wvf_ref, bvf_ref, wvt_ref, bvt_ref,
                   g_ref, bb_ref, xnew_ref, vxf_ref, vxt_ref):
    cnt = float(_B * _N)
    mean = st_ref[0:1, :] / cnt
    var = st_ref[1:2, :] / cnt - mean * mean
    scale = lax.rsqrt(var + _EPS) * g_ref[...]
    xb = (xt_ref[0] - mean) * scale + bb_ref[...]
    xn = x_ref[0] + jnp.maximum(xb, 0.0)
    xnew_ref[0] = xn
    vxf_ref[0] = _mm(xn, wvf_ref[...]) + bvf_ref[...]
    vxt_ref[0] = _mm(xn, wvt_ref[...]) + bvt_ref[...]


def _node_fin(x, x_tmp, st, wvf, bvf2, wvt, bvt2, g2, bb2):
    rows = 2000
    full = lambda b, i: (0, 0)
    return pl.pallas_call(
        _node_fin_body,
        grid=(_B, _N // rows),
        in_specs=[
            pl.BlockSpec((1, rows, _H), lambda b, i: (b, i, 0)),
            pl.BlockSpec((1, rows, _H), lambda b, i: (b, i, 0)),
            pl.BlockSpec((2, _H), full),
            pl.BlockSpec((_H, _H), full),
            pl.BlockSpec((1, _H), full),
            pl.BlockSpec((_H, _H), full),
            pl.BlockSpec((1, _H), full),
            pl.BlockSpec((1, _H), full),
            pl.BlockSpec((1, _H), full),
        ],
        out_specs=[
            pl.BlockSpec((1, rows, _H), lambda b, i: (b, i, 0)),
            pl.BlockSpec((1, rows, _H), lambda b, i: (b, i, 0)),
            pl.BlockSpec((1, rows, _H), lambda b, i: (b, i, 0)),
        ],
        out_shape=[
            jax.ShapeDtypeStruct((_B, _N, _H), jnp.float32),
            jax.ShapeDtypeStruct((_B, _N, _H), jnp.float32),
            jax.ShapeDtypeStruct((_B, _N, _H), jnp.float32),
        ],
    )(x, x_tmp, st, wvf, bvf2, wvt, bvt2, g2, bb2)


def _stage_b_body(e_ref, vxeg_ref, inve_ref, vxf_ref, idx_ref, wu_ref, bu_ref,
                  wiu_ref, biu_ref, wph_ref, etmp_ref, st_ref):
    b = pl.program_id(0)
    i = pl.program_id(1)
    ue = _mm(e_ref[0], wu_ref[...]) + bu_ref[...]
    inv_emb = _mm(inve_ref[0], wiu_ref[...]) + biu_ref[...]
    isph = idx_ref[0] == _L
    inv_emb = jnp.where(isph, wph_ref[...], inv_emb)
    vxe3 = vxeg_ref[0].reshape(_BLN, _SF, _H) + vxf_ref[0][:, None, :]
    et = ue + vxe3.reshape(_BL, _H) + inv_emb
    etmp_ref[0] = et.astype(jnp.bfloat16)

    @pl.when(jnp.logical_and(b == 0, i == 0))
    def _():
        st_ref[...] = jnp.zeros_like(st_ref)

    st_ref[0:1, :] += jnp.sum(et, axis=0, keepdims=True)
    st_ref[1:2, :] += jnp.sum(et * et, axis=0, keepdims=True)


def _stage_b(e, vxeg, inv_e, vx_from, inv3, wu, bu2, wiu, biu2, wph2):
    full = lambda b, i: (0, 0)
    return pl.pallas_call(
        _stage_b_body,
        grid=(_B, _NBE),
        in_specs=[
            pl.BlockSpec((1, _BL, _H), lambda b, i: (b, i, 0)),
            pl.BlockSpec((1, _BL, _H), lambda b, i: (b, i, 0)),
            pl.BlockSpec((1, _BL, _H), lambda b, i: (b, i, 0)),
            pl.BlockSpec((1, _BLN, _H), lambda b, i: (b, i, 0)),
            pl.BlockSpec((1, _BL, 1), lambda b, i: (b, i, 0)),
            pl.BlockSpec((_H, _H), full),
            pl.BlockSpec((1, _H), full),
            pl.BlockSpec((_H, _H), full),
            pl.BlockSpec((1, _H), full),
            pl.BlockSpec((1, _H), full),
        ],
        out_specs=[
            pl.BlockSpec((1, _BL, _H), lambda b, i: (b, i, 0)),
            pl.BlockSpec((2, _H), full),
        ],
        out_shape=[
            jax.ShapeDtypeStruct((_B, _L, _H), jnp.bfloat16),
            jax.ShapeDtypeStruct((2, _H), jnp.float32),
        ],
    )(e, vxeg, inv_e, vx_from, inv3, wu, bu2, wiu, biu2, wph2)


def _edge_fin_body(et_ref, e_ref, st_ref, g_ref, bb_ref, enew_ref):
    cnt = float(_B * _L)
    mean = st_ref[0:1, :] / cnt
    var = st_ref[1:2, :] / cnt - mean * mean
    scale = lax.rsqrt(var + _EPS) * g_ref[...]
    eb = (et_ref[0].astype(jnp.float32) - mean) * scale + bb_ref[...]
    enew_ref[0] = e_ref[0] + jnp.maximum(eb, 0.0)


def _edge_fin(e_tmp, e, st, g2, bb2):
    full = lambda b, i: (0, 0)
    return pl.pallas_call(
        _edge_fin_body,
        grid=(_B, _NBE),
        in_specs=[
            pl.BlockSpec((1, _BL, _H), lambda b, i: (b, i, 0)),
            pl.BlockSpec((1, _BL, _H), lambda b, i: (b, i, 0)),
            pl.BlockSpec((2, _H), full),
            pl.BlockSpec((1, _H), full),
            pl.BlockSpec((1, _H), full),
        ],
        out_specs=pl.BlockSpec((1, _BL, _H), lambda b, i: (b, i, 0)),
        out_shape=jax.ShapeDtypeStruct((_B, _L, _H), jnp.float32),
    )(e_tmp, e, st, g2, bb2)


def kernel(x, e, edge_index, inverse_edge_index, Wn, bn, Wt, bt, We, be,
           WU, bU, WVf, bVf, WVt, bVt, WiU, biU, W_ph,
           g_node, b_node, g_edge, b_edge):
    x = x.astype(jnp.float32)
    e = e.astype(jnp.float32)
    ei = edge_index.astype(jnp.int32)
    inv = inverse_edge_index.astype(jnp.int32)
    b2 = lambda v: v.astype(jnp.float32).reshape(1, _H)

    x_flat = x.reshape(_B * _N, _H)
    e_flat = e.reshape(_B * _L, _H)

    boff_n = (jnp.arange(_B, dtype=jnp.int32) * _N)[:, None]
    boff_l = (jnp.arange(_B, dtype=jnp.int32) * _L)[:, None]
    pad = _MP - _M
    flat_e = jnp.concatenate(
        [(ei + boff_n).reshape(-1), jnp.zeros((pad,), jnp.int32)])
    flat_inv = jnp.concatenate(
        [(jnp.minimum(inv, _L - 1) + boff_l).reshape(-1),
         jnp.zeros((pad,), jnp.int32)])

    vx = _k_vx(x_flat, Wt, b2(bt))
    vxg = _sc_gather(vx, flat_e)[:_M].reshape(_B, _L, _H)
    inv_e = _sc_gather(e_flat, flat_inv)[:_M].reshape(_B, _L, _H)
    x_tmp, st_n = _stage_a(e, vxg, x, We, b2(be), Wn, b2(bn))
    x_new, vx_from, vx_to = _node_fin(
        x, x_tmp, st_n, WVf, b2(bVf), WVt, b2(bVt), b2(g_node), b2(b_node))

    vxeg = _sc_gather(vx_to.reshape(_B * _N, _H), flat_e)[:_M].reshape(_B, _L, _H)
    e_tmp, st_e = _stage_b(
        e, vxeg, inv_e, vx_from, inv.reshape(_B, _L, 1),
        WU, b2(bU), WiU, b2(biU), b2(W_ph))
    e_new = _edge_fin(e_tmp, e, st_e, b2(g_edge), b2(b_edge))
    return (x_new, e_new)


# speedup vs baseline: 1.2685x; 1.2685x over previous
import functools

import jax
import jax.numpy as jnp
from jax import lax
from jax.experimental import pallas as pl
from jax.experimental.pallas import tpu as pltpu
from jax.experimental.pallas import tpu_sc as plsc

_B, _N, _SF, _H = 2, 10000, 20, 128
_L = _N * _SF
_EPS = 1e-5

_BL = 8000
_BLN = _BL // _SF
_NBE = _L // _BL

_NC, _NS = 2, 16
_NW = _NC * _NS
_CHUNK = 128
_M = _B * _L
_GRAN = _NW * 2 * _CHUNK
_MP = ((_M + _GRAN - 1) // _GRAN) * _GRAN


def _mm(a, w):
    return lax.dot_general(a, w, (((1,), (1,)), ((), ())),
                           preferred_element_type=jnp.float32)


def _sc_gather(table, idx):
    mp, = idx.shape
    _, h = table.shape
    dt = table.dtype
    m_per_w = mp // _NW
    nsteps = m_per_w // (2 * _CHUNK)
    mesh = plsc.VectorSubcoreMesh(core_axis_name="c", subcore_axis_name="s")

    @functools.partial(
        pl.kernel,
        mesh=mesh,
        out_type=jax.ShapeDtypeStruct((mp, h), dt),
        scratch_types=[
            pltpu.VMEM((m_per_w,), jnp.int32),
            pltpu.VMEM((_CHUNK, h), dt),
            pltpu.VMEM((_CHUNK, h), dt),
            pltpu.SemaphoreType.DMA,
            pltpu.SemaphoreType.DMA,
        ],
    )
    def k(table_hbm, idx_hbm, out_hbm, idx_v, buf0, buf1, sem0, sem1):
        wid = lax.axis_index("s") * _NC + lax.axis_index("c")
        base = wid * m_per_w
        pltpu.sync_copy(idx_hbm.at[pl.ds(base, m_per_w)], idx_v)

        def body(s, carry):
            c0 = 2 * s
            c1 = c0 + 1
            g0 = pltpu.async_copy(
                table_hbm.at[idx_v.at[pl.ds(c0 * _CHUNK, _CHUNK)]], buf0, sem0)
            g1 = pltpu.async_copy(
                table_hbm.at[idx_v.at[pl.ds(c1 * _CHUNK, _CHUNK)]], buf1, sem1)
            g0.wait()
            pltpu.sync_copy(buf0, out_hbm.at[pl.ds(base + c0 * _CHUNK, _CHUNK)])
            g1.wait()
            pltpu.sync_copy(buf1, out_hbm.at[pl.ds(base + c1 * _CHUNK, _CHUNK)])
            return carry

        lax.fori_loop(0, nsteps, body, 0)

    return k(table, idx)


def _vx_body(x_ref, w_ref, b_ref, o_ref):
    o_ref[...] = _mm(x_ref[...], w_ref[...]) + b_ref[...]


def _k_vx(x_flat, w, b2):
    rows = 2000
    return pl.pallas_call(
        _vx_body,
        grid=(x_flat.shape[0] // rows,),
        in_specs=[
            pl.BlockSpec((rows, _H), lambda i: (i, 0)),
            pl.BlockSpec((_H, _H), lambda i: (0, 0)),
            pl.BlockSpec((1, _H), lambda i: (0, 0)),
        ],
        out_specs=pl.BlockSpec((rows, _H), lambda i: (i, 0)),
        out_shape=jax.ShapeDtypeStruct(x_flat.shape, jnp.float32),
    )(x_flat, w, b2)


def _stage_a_body(e_ref, vxg_ref, x_ref, we_ref, be_ref, wn_ref, bn_ref,
                  xtmp_ref, st_ref):
    b = pl.program_id(0)
    i = pl.program_id(1)
    ve = _mm(e_ref[0], we_ref[...]) + be_ref[...]
    ve3 = ve.reshape(_BLN, _SF, _H)
    m = jnp.max(ve3, axis=1, keepdims=True)
    p = jnp.exp(ve3 - m)
    s = jnp.sum(p, axis=1)
    vxg3 = vxg_ref[...].reshape(_BLN, _SF, _H)
    to = jnp.sum(p * vxg3, axis=1) / s
    xt = _mm(x_ref[0], wn_ref[...]) + bn_ref[...] + to
    xtmp_ref[0] = xt

    @pl.when(jnp.logical_and(b == 0, i == 0))
    def _():
        st_ref[...] = jnp.zeros_like(st_ref)

    st_ref[0:1, :] += jnp.sum(xt, axis=0, keepdims=True)
    st_ref[1:2, :] += jnp.sum(xt * xt, axis=0, keepdims=True)


def _stage_a(e, vxg, x, we, be2, wn, bn2):
    return pl.pallas_call(
        _stage_a_body,
        grid=(_B, _NBE),
        in_specs=[
            pl.BlockSpec((1, _BL, _H), lambda b, i: (b, i, 0)),
            pl.BlockSpec((_BL, _H), lambda b, i: (b * _NBE + i, 0)),
            pl.BlockSpec((1, _BLN, _H), lambda b, i: (b, i, 0)),
            pl.BlockSpec((_H, _H), lambda b, i: (0, 0)),
            pl.BlockSpec((1, _H), lambda b, i: (0, 0)),
            pl.BlockSpec((_H, _H), lambda b, i: (0, 0)),
            pl.BlockSpec((1, _H), lambda b, i: (0, 0)),
        ],
        out_specs=[
            pl.BlockSpec((1, _BLN, _H), lambda b, i: (b, i, 0)),
            pl.BlockSpec((2, _H), lambda b, i: (0, 0)),
        ],
        out_shape=[
            jax.ShapeDtypeStruct((_B, _N, _H), jnp.float32),
            jax.ShapeDtypeStruct((2, _H), jnp.float32),
        ],
    )(e, vxg, x, we, be2, wn, bn2)


def _node_fin_body(x_ref, xt_ref, st_ref, wvf_ref, bvf_ref, wvt_ref, bvt_ref,
                   g_ref, bb_ref, xnew_ref, vxf_ref, vxt_ref):
    cnt = float(_B * _N)
    mean = st_ref[0:1, :] / cnt
    var = st_ref[1:2, :] / cnt - mean * mean
    scale = lax.rsqrt(var + _EPS) * g_ref[...]
    xb = (xt_ref[0] - mean) * scale + bb_ref[...]
    xn = x_ref[0] + jnp.maximum(xb, 0.0)
    xnew_ref[0] = xn
    vxf_ref[0] = _mm(xn, wvf_ref[...]) + bvf_ref[...]
    vxt_ref[0] = _mm(xn, wvt_ref[...]) + bvt_ref[...]


def _node_fin(x, x_tmp, st, wvf, bvf2, wvt, bvt2, g2, bb2):
    rows = 2000
    full = lambda b, i: (0, 0)
    return pl.pallas_call(
        _node_fin_body,
        grid=(_B, _N // rows),
        in_specs=[
            pl.BlockSpec((1, rows, _H), lambda b, i: (b, i, 0)),
            pl.BlockSpec((1, rows, _H), lambda b, i: (b, i, 0)),
            pl.BlockSpec((2, _H), full),
            pl.BlockSpec((_H, _H), full),
            pl.BlockSpec((1, _H), full),
            pl.BlockSpec((_H, _H), full),
            pl.BlockSpec((1, _H), full),
            pl.BlockSpec((1, _H), full),
            pl.BlockSpec((1, _H), full),
        ],
        out_specs=[
            pl.BlockSpec((1, rows, _H), lambda b, i: (b, i, 0)),
            pl.BlockSpec((1, rows, _H), lambda b, i: (b, i, 0)),
            pl.BlockSpec((1, rows, _H), lambda b, i: (b, i, 0)),
        ],
        out_shape=[
            jax.ShapeDtypeStruct((_B, _N, _H), jnp.float32),
            jax.ShapeDtypeStruct((_B, _N, _H), jnp.float32),
            jax.ShapeDtypeStruct((_B, _N, _H), jnp.float32),
        ],
    )(x, x_tmp, st, wvf, bvf2, wvt, bvt2, g2, bb2)


def _stage_b_body(e_ref, vxeg_ref, inve_ref, vxf_ref, idx_ref, wu_ref, bu_ref,
                  wiu_ref, biu_ref, wph_ref, etmp_ref, st_ref):
    b = pl.program_id(0)
    i = pl.program_id(1)
    ue = _mm(e_ref[0], wu_ref[...]) + bu_ref[...]
    inv_emb = _mm(inve_ref[...], wiu_ref[...]) + biu_ref[...]
    isph = idx_ref[0] == _L
    inv_emb = jnp.where(isph, wph_ref[...], inv_emb)
    vxe3 = vxeg_ref[...].reshape(_BLN, _SF, _H) + vxf_ref[0][:, None, :]
    et = ue + vxe3.reshape(_BL, _H) + inv_emb
    etmp_ref[0] = et.astype(jnp.bfloat16)

    @pl.when(jnp.logical_and(b == 0, i == 0))
    def _():
        st_ref[...] = jnp.zeros_like(st_ref)

    st_ref[0:1, :] += jnp.sum(et, axis=0, keepdims=True)
    st_ref[1:2, :] += jnp.sum(et * et, axis=0, keepdims=True)


def _stage_b(e, vxeg, inv_e, vx_from, inv3, wu, bu2, wiu, biu2, wph2):
    full = lambda b, i: (0, 0)
    return pl.pallas_call(
        _stage_b_body,
        grid=(_B, _NBE),
        in_specs=[
            pl.BlockSpec((1, _BL, _H), lambda b, i: (b, i, 0)),
            pl.BlockSpec((_BL, _H), lambda b, i: (b * _NBE + i, 0)),
            pl.BlockSpec((_BL, _H), lambda b, i: (b * _NBE + i, 0)),
            pl.BlockSpec((1, _BLN, _H), lambda b, i: (b, i, 0)),
            pl.BlockSpec((1, _BL, 1), lambda b, i: (b, i, 0)),
            pl.BlockSpec((_H, _H), full),
            pl.BlockSpec((1, _H), full),
            pl.BlockSpec((_H, _H), full),
            pl.BlockSpec((1, _H), full),
            pl.BlockSpec((1, _H), full),
        ],
        out_specs=[
            pl.BlockSpec((1, _BL, _H), lambda b, i: (b, i, 0)),
            pl.BlockSpec((2, _H), full),
        ],
        out_shape=[
            jax.ShapeDtypeStruct((_B, _L, _H), jnp.bfloat16),
            jax.ShapeDtypeStruct((2, _H), jnp.float32),
        ],
    )(e, vxeg, inv_e, vx_from, inv3, wu, bu2, wiu, biu2, wph2)


def _edge_fin_body(et_ref, e_ref, st_ref, g_ref, bb_ref, enew_ref):
    cnt = float(_B * _L)
    mean = st_ref[0:1, :] / cnt
    var = st_ref[1:2, :] / cnt - mean * mean
    scale = lax.rsqrt(var + _EPS) * g_ref[...]
    eb = (et_ref[0].astype(jnp.float32) - mean) * scale + bb_ref[...]
    enew_ref[0] = e_ref[0] + jnp.maximum(eb, 0.0)


def _edge_fin(e_tmp, e, st, g2, bb2):
    full = lambda b, i: (0, 0)
    return pl.pallas_call(
        _edge_fin_body,
        grid=(_B, _NBE),
        in_specs=[
            pl.BlockSpec((1, _BL, _H), lambda b, i: (b, i, 0)),
            pl.BlockSpec((1, _BL, _H), lambda b, i: (b, i, 0)),
            pl.BlockSpec((2, _H), full),
            pl.BlockSpec((1, _H), full),
            pl.BlockSpec((1, _H), full),
        ],
        out_specs=pl.BlockSpec((1, _BL, _H), lambda b, i: (b, i, 0)),
        out_shape=jax.ShapeDtypeStruct((_B, _L, _H), jnp.float32),
    )(e_tmp, e, st, g2, bb2)


def kernel(x, e, edge_index, inverse_edge_index, Wn, bn, Wt, bt, We, be,
           WU, bU, WVf, bVf, WVt, bVt, WiU, biU, W_ph,
           g_node, b_node, g_edge, b_edge):
    x = x.astype(jnp.float32)
    e = e.astype(jnp.float32)
    ei = edge_index.astype(jnp.int32)
    inv = inverse_edge_index.astype(jnp.int32)
    b2 = lambda v: v.astype(jnp.float32).reshape(1, _H)

    x_flat = x.reshape(_B * _N, _H)
    e_flat = e.reshape(_B * _L, _H)

    boff_n = (jnp.arange(_B, dtype=jnp.int32) * _N)[:, None]
    boff_l = (jnp.arange(_B, dtype=jnp.int32) * _L)[:, None]
    pad = _MP - _M
    flat_e = jnp.concatenate(
        [(ei + boff_n).reshape(-1), jnp.zeros((pad,), jnp.int32)])
    flat_inv = jnp.concatenate(
        [(jnp.minimum(inv, _L - 1) + boff_l).reshape(-1),
         jnp.zeros((pad,), jnp.int32)])

    vx = _k_vx(x_flat, Wt, b2(bt))
    vxg = _sc_gather(vx, flat_e)
    inv_e = _sc_gather(e_flat, flat_inv)
    x_tmp, st_n = _stage_a(e, vxg, x, We, b2(be), Wn, b2(bn))
    x_new, vx_from, vx_to = _node_fin(
        x, x_tmp, st_n, WVf, b2(bVf), WVt, b2(bVt), b2(g_node), b2(b_node))

    vxeg = _sc_gather(vx_to.reshape(_B * _N, _H), flat_e)
    e_tmp, st_e = _stage_b(
        e, vxeg, inv_e, vx_from, inv.reshape(_B, _L, 1),
        WU, b2(bU), WiU, b2(biU), b2(W_ph))
    e_new = _edge_fin(e_tmp, e, st_e, b2(g_edge), b2(b_edge))
    return (x_new, e_new)
